# baseline (device time: 10402 ns/iter reference)
import jax
import jax.numpy as jnp
from jax import lax
from jax.experimental import pallas as pl
from jax.experimental.pallas import tpu as pltpu

N_DEV = 4
C_GLOBAL = 512.0
EPS = 1e-5


def kernel(x, t_emb, W_scale, W_shift):
    b, s, c = x.shape

    def body(x_ref, t_ref, wsc_ref, wsh_ref, out_ref,
             my_stats, comm_ref, send_sems, recv_sems):
        my = lax.axis_index("i")

        xv = x_ref[...]
        my_stats[0] = jnp.sum(xv, axis=2)
        my_stats[1] = jnp.sum(xv * xv, axis=2)

        barrier_sem = pltpu.get_barrier_semaphore()
        for d in range(1, N_DEV):
            pl.semaphore_signal(
                barrier_sem, inc=1,
                device_id=((my + d) % N_DEV,),
                device_id_type=pl.DeviceIdType.MESH,
            )
        pl.semaphore_wait(barrier_sem, N_DEV - 1)

        rdmas = []
        for d in (2, 1, 3):
            rdma = pltpu.make_async_remote_copy(
                src_ref=my_stats,
                dst_ref=comm_ref.at[d - 1],
                send_sem=send_sems.at[d - 1],
                recv_sem=recv_sems.at[d - 1],
                device_id=((my + d) % N_DEV,),
                device_id_type=pl.DeviceIdType.MESH,
            )
            rdma.start()
            rdmas.append(rdma)

        tv = t_ref[...]
        scale = jnp.dot(tv, wsc_ref[...], preferred_element_type=jnp.float32)
        shift = jnp.dot(tv, wsh_ref[...], preferred_element_type=jnp.float32)

        for rdma in rdmas:
            rdma.wait_recv()

        tot = my_stats[...] + comm_ref[0] + comm_ref[1] + comm_ref[2]
        mean = tot[0] * (1.0 / C_GLOBAL)
        var = tot[1] * (1.0 / C_GLOBAL) - mean * mean
        rstd = lax.rsqrt(var + EPS)
        h = (xv - mean[:, :, None]) * rstd[:, :, None]
        out_ref[...] = h * (1.0 + scale[:, None, :]) + shift[:, None, :]

        for rdma in rdmas:
            rdma.wait_send()

    return pl.pallas_call(
        body,
        out_shape=jax.ShapeDtypeStruct((b, s, c), jnp.float32),
        in_specs=[pl.BlockSpec(memory_space=pltpu.VMEM)] * 4,
        out_specs=pl.BlockSpec(memory_space=pltpu.VMEM),
        scratch_shapes=[
            pltpu.VMEM((2, b, s), jnp.float32),
            pltpu.VMEM((N_DEV - 1, 2, b, s), jnp.float32),
            pltpu.SemaphoreType.DMA((N_DEV - 1,)),
            pltpu.SemaphoreType.DMA((N_DEV - 1,)),
        ],
        compiler_params=pltpu.CompilerParams(collective_id=0),
    )(x, t_emb, W_scale, W_shift)


# device time: 10125 ns/iter; 1.0274x vs baseline; 1.0274x over previous
import jax
import jax.numpy as jnp
from jax import lax
from jax.experimental import pallas as pl
from jax.experimental.pallas import tpu as pltpu

N_DEV = 4
C_GLOBAL = 512.0
EPS = 1e-5


def kernel(x, t_emb, W_scale, W_shift):
    b, s, c = x.shape

    def body(x_hbm, t_ref, wsc_ref, wsh_ref, out_ref,
             xv_ref, my_stats, comm_ref, load_sem, send_sems, recv_sems):
        my = lax.axis_index("i")

        barrier_sem = pltpu.get_barrier_semaphore()
        for d in range(1, N_DEV):
            pl.semaphore_signal(
                barrier_sem, inc=1,
                device_id=((my + d) % N_DEV,),
                device_id_type=pl.DeviceIdType.MESH,
            )

        load = pltpu.make_async_copy(x_hbm, xv_ref, load_sem)
        load.start()

        tv = t_ref[...]
        scale = jnp.dot(tv, wsc_ref[...], preferred_element_type=jnp.float32)
        shift = jnp.dot(tv, wsh_ref[...], preferred_element_type=jnp.float32)

        load.wait()
        xv = xv_ref[...]
        my_stats[0] = jnp.sum(xv, axis=2)
        my_stats[1] = jnp.sum(xv * xv, axis=2)

        pl.semaphore_wait(barrier_sem, N_DEV - 1)

        rdmas = []
        for d in (2, 1, 3):
            rdma = pltpu.make_async_remote_copy(
                src_ref=my_stats,
                dst_ref=comm_ref.at[d - 1],
                send_sem=send_sems.at[d - 1],
                recv_sem=recv_sems.at[d - 1],
                device_id=((my + d) % N_DEV,),
                device_id_type=pl.DeviceIdType.MESH,
            )
            rdma.start()
            rdmas.append(rdma)

        for rdma in rdmas:
            rdma.wait_recv()

        tot = my_stats[...] + comm_ref[0] + comm_ref[1] + comm_ref[2]
        mean = tot[0] * (1.0 / C_GLOBAL)
        var = tot[1] * (1.0 / C_GLOBAL) - mean * mean
        rstd = lax.rsqrt(var + EPS)
        h = (xv - mean[:, :, None]) * rstd[:, :, None]
        out_ref[...] = h * (1.0 + scale[:, None, :]) + shift[:, None, :]

        for rdma in rdmas:
            rdma.wait_send()

    return pl.pallas_call(
        body,
        out_shape=jax.ShapeDtypeStruct((b, s, c), jnp.float32),
        in_specs=[
            pl.BlockSpec(memory_space=pl.ANY),
            pl.BlockSpec(memory_space=pltpu.VMEM),
            pl.BlockSpec(memory_space=pltpu.VMEM),
            pl.BlockSpec(memory_space=pltpu.VMEM),
        ],
        out_specs=pl.BlockSpec(memory_space=pltpu.VMEM),
        scratch_shapes=[
            pltpu.VMEM((b, s, c), jnp.float32),
            pltpu.VMEM((2, b, s), jnp.float32),
            pltpu.VMEM((N_DEV - 1, 2, b, s), jnp.float32),
            pltpu.SemaphoreType.DMA,
            pltpu.SemaphoreType.DMA((N_DEV - 1,)),
            pltpu.SemaphoreType.DMA((N_DEV - 1,)),
        ],
        compiler_params=pltpu.CompilerParams(collective_id=0),
    )(x, t_emb, W_scale, W_shift)


# device time: 9987 ns/iter; 1.0416x vs baseline; 1.0138x over previous
import jax
import jax.numpy as jnp
from jax import lax
from jax.experimental import pallas as pl
from jax.experimental.pallas import tpu as pltpu

N_DEV = 4
N_HALF = 2
C_GLOBAL = 512.0
EPS = 1e-5


def kernel(x, t_emb, W_scale, W_shift):
    b, s, c = x.shape
    sh = s // N_HALF

    def body(x_hbm, t_ref, wsc_ref, wsh_ref, out_hbm,
             xv_ref, out_vmem, my_stats, comm_ref,
             load_sem, store_sems, send_sems, recv_sems):
        my = lax.axis_index("i")

        barrier_sem = pltpu.get_barrier_semaphore()
        for d in range(1, N_DEV):
            pl.semaphore_signal(
                barrier_sem, inc=1,
                device_id=((my + d) % N_DEV,),
                device_id_type=pl.DeviceIdType.MESH,
            )

        load = pltpu.make_async_copy(x_hbm, xv_ref, load_sem)
        load.start()
        load.wait()
        xv = xv_ref[...]

        rdmas = []
        for h in range(N_HALF):
            xh = xv[:, h * sh:(h + 1) * sh, :]
            my_stats[h, 0] = jnp.sum(xh, axis=2)
            my_stats[h, 1] = jnp.sum(xh * xh, axis=2)
            if h == 0:
                pl.semaphore_wait(barrier_sem, N_DEV - 1)
            for d in (2, 1, 3):
                k = (d - 1) * N_HALF + h
                rdma = pltpu.make_async_remote_copy(
                    src_ref=my_stats.at[h],
                    dst_ref=comm_ref.at[k],
                    send_sem=send_sems.at[k],
                    recv_sem=recv_sems.at[k],
                    device_id=((my + d) % N_DEV,),
                    device_id_type=pl.DeviceIdType.MESH,
                )
                rdma.start()
                rdmas.append((h, d, rdma))

        tv = t_ref[...]
        scale = jnp.dot(tv, wsc_ref[...], preferred_element_type=jnp.float32)
        shift = jnp.dot(tv, wsh_ref[...], preferred_element_type=jnp.float32)
        g = 1.0 + scale

        stores = []
        for h in range(N_HALF):
            for _, d, rdma in rdmas:
                if _ == h:
                    rdma.wait_recv()
            tot = my_stats[h]
            for d in range(1, N_DEV):
                tot = tot + comm_ref[(d - 1) * N_HALF + h]
            mean = tot[0] * (1.0 / C_GLOBAL)
            var = tot[1] * (1.0 / C_GLOBAL) - mean * mean
            rstd = lax.rsqrt(var + EPS)
            xh = xv[:, h * sh:(h + 1) * sh, :]
            hn = (xh - mean[:, :, None]) * rstd[:, :, None]
            out_vmem[h] = hn * g[:, None, :] + shift[:, None, :]
            store = pltpu.make_async_copy(
                out_vmem.at[h],
                out_hbm.at[:, pl.ds(h * sh, sh), :],
                store_sems.at[h],
            )
            store.start()
            stores.append(store)

        for store in stores:
            store.wait()
        for _, _, rdma in rdmas:
            rdma.wait_send()

    return pl.pallas_call(
        body,
        out_shape=jax.ShapeDtypeStruct((b, s, c), jnp.float32),
        in_specs=[
            pl.BlockSpec(memory_space=pl.ANY),
            pl.BlockSpec(memory_space=pltpu.VMEM),
            pl.BlockSpec(memory_space=pltpu.VMEM),
            pl.BlockSpec(memory_space=pltpu.VMEM),
        ],
        out_specs=pl.BlockSpec(memory_space=pl.ANY),
        scratch_shapes=[
            pltpu.VMEM((b, s, c), jnp.float32),
            pltpu.VMEM((N_HALF, b, sh, c), jnp.float32),
            pltpu.VMEM((N_HALF, 2, b, sh), jnp.float32),
            pltpu.VMEM(((N_DEV - 1) * N_HALF, 2, b, sh), jnp.float32),
            pltpu.SemaphoreType.DMA,
            pltpu.SemaphoreType.DMA((N_HALF,)),
            pltpu.SemaphoreType.DMA(((N_DEV - 1) * N_HALF,)),
            pltpu.SemaphoreType.DMA(((N_DEV - 1) * N_HALF,)),
        ],
        compiler_params=pltpu.CompilerParams(collective_id=0),
    )(x, t_emb, W_scale, W_shift)


# device time: 9838 ns/iter; 1.0573x vs baseline; 1.0151x over previous
import jax
import jax.numpy as jnp
from jax import lax
from jax.experimental import pallas as pl
from jax.experimental.pallas import tpu as pltpu

N_DEV = 4
N_HALF = 2
C_GLOBAL = 512.0
EPS = 1e-5


def kernel(x, t_emb, W_scale, W_shift):
    b, s, c = x.shape
    sh = s // N_HALF

    def body(x_hbm, t_ref, wsc_ref, wsh_ref, out_hbm,
             xv_ref, out_vmem, my_stats, comm_ref,
             load_sems, store_sems, send_sems, recv_sems):
        my = lax.axis_index("i")

        barrier_sem = pltpu.get_barrier_semaphore()
        for d in range(1, N_DEV):
            pl.semaphore_signal(
                barrier_sem, inc=1,
                device_id=((my + d) % N_DEV,),
                device_id_type=pl.DeviceIdType.MESH,
            )

        loads = []
        for h in range(N_HALF):
            ld = pltpu.make_async_copy(
                x_hbm.at[:, pl.ds(h * sh, sh), :],
                xv_ref.at[:, pl.ds(h * sh, sh), :],
                load_sems.at[h],
            )
            ld.start()
            loads.append(ld)

        rdmas = []
        for h in range(N_HALF):
            loads[h].wait()
            xh = xv_ref[:, h * sh:(h + 1) * sh, :]
            my_stats[h, 0] = jnp.sum(xh, axis=2)
            my_stats[h, 1] = jnp.sum(xh * xh, axis=2)
            if h == 0:
                pl.semaphore_wait(barrier_sem, N_DEV - 1)
            for d in (2, 1, 3):
                k = (d - 1) * N_HALF + h
                rdma = pltpu.make_async_remote_copy(
                    src_ref=my_stats.at[h],
                    dst_ref=comm_ref.at[k],
                    send_sem=send_sems.at[k],
                    recv_sem=recv_sems.at[k],
                    device_id=((my + d) % N_DEV,),
                    device_id_type=pl.DeviceIdType.MESH,
                )
                rdma.start()
                rdmas.append((h, d, rdma))

        tv = t_ref[...]
        scale = jnp.dot(tv, wsc_ref[...], preferred_element_type=jnp.float32)
        shift = jnp.dot(tv, wsh_ref[...], preferred_element_type=jnp.float32)
        g = 1.0 + scale

        stores = []
        for h in range(N_HALF):
            for _, d, rdma in rdmas:
                if _ == h:
                    rdma.wait_recv()
            tot = my_stats[h]
            for d in range(1, N_DEV):
                tot = tot + comm_ref[(d - 1) * N_HALF + h]
            mean = tot[0] * (1.0 / C_GLOBAL)
            var = tot[1] * (1.0 / C_GLOBAL) - mean * mean
            rstd = lax.rsqrt(var + EPS)
            xh = xv_ref[:, h * sh:(h + 1) * sh, :]
            hn = (xh - mean[:, :, None]) * rstd[:, :, None]
            out_vmem[h] = hn * g[:, None, :] + shift[:, None, :]
            store = pltpu.make_async_copy(
                out_vmem.at[h],
                out_hbm.at[:, pl.ds(h * sh, sh), :],
                store_sems.at[h],
            )
            store.start()
            stores.append(store)

        for store in stores:
            store.wait()
        for _, _, rdma in rdmas:
            rdma.wait_send()

    return pl.pallas_call(
        body,
        out_shape=jax.ShapeDtypeStruct((b, s, c), jnp.float32),
        in_specs=[
            pl.BlockSpec(memory_space=pl.ANY),
            pl.BlockSpec(memory_space=pltpu.VMEM),
            pl.BlockSpec(memory_space=pltpu.VMEM),
            pl.BlockSpec(memory_space=pltpu.VMEM),
        ],
        out_specs=pl.BlockSpec(memory_space=pl.ANY),
        scratch_shapes=[
            pltpu.VMEM((b, s, c), jnp.float32),
            pltpu.VMEM((N_HALF, b, sh, c), jnp.float32),
            pltpu.VMEM((N_HALF, 2, b, sh), jnp.float32),
            pltpu.VMEM(((N_DEV - 1) * N_HALF, 2, b, sh), jnp.float32),
            pltpu.SemaphoreType.DMA((N_HALF,)),
            pltpu.SemaphoreType.DMA((N_HALF,)),
            pltpu.SemaphoreType.DMA(((N_DEV - 1) * N_HALF,)),
            pltpu.SemaphoreType.DMA(((N_DEV - 1) * N_HALF,)),
        ],
        compiler_params=pltpu.CompilerParams(collective_id=0),
    )(x, t_emb, W_scale, W_shift)
